# baseline (speedup 1.0000x reference)
import functools

import jax
import jax.numpy as jnp
from jax import lax
from jax.experimental import pallas as pl
from jax.experimental.pallas import tpu as pltpu
from jax.experimental.pallas import tpu_sc as plsc

_NUM_CORES = 2
_NUM_SUBCORES = 16
_NUM_WORKERS = _NUM_CORES * _NUM_SUBCORES
_LANES = 16


def _proj_body(e_ref, w_ref, o_ref):
    o_ref[...] = jnp.dot(e_ref[...], w_ref[...],
                         precision=jax.lax.Precision.HIGHEST,
                         preferred_element_type=jnp.float32)


def _project(vocab, dim, pdim):
    bm = 8000
    assert vocab % bm == 0
    return pl.pallas_call(
        _proj_body,
        grid=(vocab // bm,),
        in_specs=[
            pl.BlockSpec((bm, dim), lambda i: (i, 0)),
            pl.BlockSpec((dim, pdim), lambda i: (0, 0)),
        ],
        out_specs=pl.BlockSpec((bm, pdim), lambda i: (i, 0)),
        out_shape=jax.ShapeDtypeStruct((vocab, pdim), jnp.float32),
    )


def _make_pooled(seq, batch, pdim):
    bpw = batch // _NUM_WORKERS
    mesh = plsc.VectorSubcoreMesh(core_axis_name="c", subcore_axis_name="s")

    @functools.partial(
        pl.kernel,
        mesh=mesh,
        out_type=jax.ShapeDtypeStruct((batch, pdim), jnp.float32),
        compiler_params=pltpu.CompilerParams(use_tc_tiling_on_sc=False),
        scratch_types=[
            pltpu.VMEM((seq, bpw), jnp.int32),
            pltpu.VMEM((bpw, pdim), jnp.float32),
            pltpu.VMEM((bpw, pdim), jnp.float32),
            pltpu.VMEM((bpw, pdim), jnp.float32),
            pltpu.VMEM((_LANES,), jnp.float32),
            pltpu.SemaphoreType.DMA,
            pltpu.SemaphoreType.DMA,
        ],
    )
    def pooled(text_hbm, ew_hbm, bias_hbm, out_hbm, idx_v, rows0, rows1,
               acc_v, bias_v, sem0, sem1):
        w = lax.axis_index("s") * _NUM_CORES + lax.axis_index("c")
        b0 = w * bpw

        pltpu.sync_copy(text_hbm.at[:, pl.ds(b0, bpw)], idx_v)
        pltpu.sync_copy(bias_hbm, bias_v)

        def gather(s, buf, sem):
            return pltpu.make_async_copy(ew_hbm.at[idx_v.at[s]], buf, sem)

        def accumulate(buf):
            @pl.loop(0, bpw, step=4)
            def _(i):
                for d in range(4):
                    acc_v[i + d, :] = acc_v[i + d, :] + buf[i + d, :]

        @pl.loop(0, bpw, step=4)
        def _(i):
            for d in range(4):
                acc_v[i + d, :] = jnp.zeros((_LANES,), jnp.float32)

        gather(0, rows0, sem0).start()

        @pl.loop(0, seq, step=2)
        def _(s):
            gather(s, rows0, sem0).wait()
            gather(s + 1, rows1, sem1).start()
            accumulate(rows0)
            gather(s + 1, rows1, sem1).wait()

            @pl.when(s + 2 < seq)
            def _():
                gather(s + 2, rows0, sem0).start()

            accumulate(rows1)

        @pl.loop(0, bpw, step=4)
        def _(i):
            for d in range(4):
                acc_v[i + d, :] = acc_v[i + d, :] + bias_v[:]

        pltpu.sync_copy(acc_v, out_hbm.at[pl.ds(b0, bpw)])

    return pooled


def kernel(text, embedding_table, fc_w, fc_b):
    seq, batch = text.shape
    vocab, dim = embedding_table.shape
    out_dim = fc_w.shape[0]
    pdim = _LANES

    wp = jnp.zeros((dim, pdim), jnp.float32).at[:, :out_dim].set(fc_w.T / seq)
    bias16 = jnp.zeros((pdim,), jnp.float32).at[:out_dim].set(fc_b)

    ew = _project(vocab, dim, pdim)(embedding_table, wp)
    pooled = _make_pooled(seq, batch, pdim)(text, ew, bias16)
    return pooled[:, :out_dim]
